# baseline (speedup 1.0000x reference)
import functools

import jax
import jax.numpy as jnp
from jax import lax
from jax.experimental import pallas as pl
from jax.experimental.pallas import tpu as pltpu
from jax.experimental.pallas import tpu_sc as plsc

_D = 64
_SCALE = 8.0
_NC, _NS = 2, 16
_NW = _NC * _NS
_B = 819200
_CHUNK = 512
_NBUF = 2
_SB = 25
_NBB = 32


def _sc_gather(xs, table):
    b_per_w = _B // _NW
    nchunks = b_per_w // _CHUNK

    mesh = plsc.VectorSubcoreMesh(core_axis_name="c", subcore_axis_name="s")

    @functools.partial(
        pl.kernel,
        out_type=jax.ShapeDtypeStruct((_B, _D), jnp.float32),
        mesh=mesh,
        scratch_types=[
            pltpu.VMEM((b_per_w,), jnp.int32),
            pltpu.VMEM((_NBUF, _CHUNK, _D), jnp.float32),
            pltpu.SemaphoreType.DMA,
            pltpu.SemaphoreType.DMA,
            pltpu.SemaphoreType.DMA,
            pltpu.SemaphoreType.DMA,
        ],
        compiler_params=pltpu.CompilerParams(use_tc_tiling_on_sc=False),
    )
    def emb(idx_hbm, table_hbm, out_hbm, idx_v, rows_v, g0, g1, s0, s1):
        wid = lax.axis_index("s") * _NC + lax.axis_index("c")
        base = wid * b_per_w
        pltpu.sync_copy(idx_hbm.at[pl.ds(base, b_per_w)], idx_v)

        gsem = (g0, g1)
        ssem = (s0, s1)

        def start_gather(slot, c):
            pltpu.make_async_copy(
                table_hbm.at[idx_v.at[pl.ds(c * _CHUNK, _CHUNK)]],
                rows_v.at[slot],
                gsem[slot],
            ).start()

        def wait_gather(slot):
            pltpu.make_async_copy(
                table_hbm.at[idx_v.at[pl.ds(0, _CHUNK)]],
                rows_v.at[slot],
                gsem[slot],
            ).wait()

        def start_scatter(slot, c):
            pltpu.make_async_copy(
                rows_v.at[slot],
                out_hbm.at[pl.ds(base + c * _CHUNK, _CHUNK)],
                ssem[slot],
            ).start()

        def wait_scatter(slot):
            pltpu.make_async_copy(
                rows_v.at[slot],
                out_hbm.at[pl.ds(base, _CHUNK)],
                ssem[slot],
            ).wait()

        start_gather(0, 0)
        start_gather(1, 1)

        @pl.loop(0, (nchunks - 2) // 2)
        def _(i):
            c0 = i * 2
            for b in range(_NBUF):
                wait_gather(b)
                start_scatter(b, c0 + b)
            for b in range(_NBUF):
                wait_scatter(b)
                start_gather(b, c0 + 2 + b)

        for b in range(_NBUF):
            wait_gather(b)
            start_scatter(b, nchunks - 2 + b)
        for b in range(_NBUF):
            wait_scatter(b)

    return emb(xs, table)


def _tc_body(rows_ref, out_ref):
    for sr in range(8):
        blk = rows_ref[0, sr * 128 : (sr + 1) * 128, :]
        t = jnp.transpose(blk, (1, 0)) * _SCALE
        out_ref[sr, :, 0, :, :] = t.reshape(8, 8, 128)


def _tc_transpose(rows3):
    return pl.pallas_call(
        _tc_body,
        grid=(_SB, _NBB),
        in_specs=[
            pl.BlockSpec((1, 1024, _D), lambda sb, bb: (sb * _NBB + bb, 0, 0)),
        ],
        out_specs=pl.BlockSpec(
            (8, 8, 1, 8, 128), lambda sb, bb: (sb, 0, bb, 0, 0)
        ),
        out_shape=jax.ShapeDtypeStruct((200, 8, _NBB, 8, 128), jnp.float32),
        compiler_params=pltpu.CompilerParams(
            dimension_semantics=("arbitrary", "arbitrary")
        ),
    )(rows3)


@jax.jit
def _lookup(xs, table):
    rows = _sc_gather(xs, table)
    return _tc_transpose(rows.reshape(_SB * _NBB, 1024, _D))


def kernel(x, table):
    xs = x.astype(jnp.int32).reshape(_NBB, 128, _SB, 8).transpose(2, 0, 3, 1)
    out5 = _lookup(xs.reshape(-1), table)
    return out5.transpose(2, 4, 0, 1, 3).reshape(4096, 200, _D)


# speedup vs baseline: 1.3259x; 1.3259x over previous
import functools

import jax
import jax.numpy as jnp
from jax import lax
from jax.experimental import pallas as pl
from jax.experimental.pallas import tpu as pltpu
from jax.experimental.pallas import tpu_sc as plsc

_D = 64
_SCALE = 8.0
_NC, _NS = 2, 16
_NW = _NC * _NS
_B = 819200
_CHUNK = 512
_NBUF = 2
_SB = 25
_NBB = 32


def _sc_gather(xs, table):
    b_per_w = _B // _NW
    nchunks = b_per_w // _CHUNK

    mesh = plsc.VectorSubcoreMesh(core_axis_name="c", subcore_axis_name="s")

    @functools.partial(
        pl.kernel,
        out_type=jax.ShapeDtypeStruct((_B, _D), jnp.float32),
        mesh=mesh,
        scratch_types=[
            pltpu.VMEM((b_per_w,), jnp.int32),
            pltpu.VMEM((_NBUF, _CHUNK, _D), jnp.float32),
            pltpu.SemaphoreType.DMA,
            pltpu.SemaphoreType.DMA,
            pltpu.SemaphoreType.DMA,
            pltpu.SemaphoreType.DMA,
        ],
        compiler_params=pltpu.CompilerParams(use_tc_tiling_on_sc=False),
    )
    def emb(idx_hbm, table_hbm, out_hbm, idx_v, rows_v, g0, g1, s0, s1):
        wid = lax.axis_index("s") * _NC + lax.axis_index("c")
        base = wid * b_per_w
        pltpu.sync_copy(idx_hbm.at[pl.ds(base, b_per_w)], idx_v)

        gsem = (g0, g1)
        ssem = (s0, s1)

        def start_gather(slot, c):
            pltpu.make_async_copy(
                table_hbm.at[idx_v.at[pl.ds(c * _CHUNK, _CHUNK)]],
                rows_v.at[slot],
                gsem[slot],
            ).start()

        def wait_gather(slot):
            pltpu.make_async_copy(
                table_hbm.at[idx_v.at[pl.ds(0, _CHUNK)]],
                rows_v.at[slot],
                gsem[slot],
            ).wait()

        def start_scatter(slot, c):
            pltpu.make_async_copy(
                rows_v.at[slot],
                out_hbm.at[pl.ds(base + c * _CHUNK, _CHUNK)],
                ssem[slot],
            ).start()

        def wait_scatter(slot):
            pltpu.make_async_copy(
                rows_v.at[slot],
                out_hbm.at[pl.ds(base, _CHUNK)],
                ssem[slot],
            ).wait()

        start_gather(0, 0)
        start_gather(1, 1)

        @pl.loop(0, (nchunks - 2) // 2)
        def _(i):
            c0 = i * 2
            for b in range(_NBUF):
                wait_gather(b)
                start_scatter(b, c0 + b)
            for b in range(_NBUF):
                wait_scatter(b)
                start_gather(b, c0 + 2 + b)

        for b in range(_NBUF):
            wait_gather(b)
            start_scatter(b, nchunks - 2 + b)
        for b in range(_NBUF):
            wait_scatter(b)

    return emb(xs, table)


def _tc_body(rows_ref, out_ref):
    for k in range(8):
        pt = jnp.transpose(rows_ref[k], (1, 0)) * _SCALE
        out_ref[k, :, 0, :, :] = pt[:_D, :].reshape(8, 8, 128)
        out_ref[k, :, 1, :, :] = pt[_D:, :].reshape(8, 8, 128)


def _tc_transpose(rows3):
    return pl.pallas_call(
        _tc_body,
        grid=(_SB, _NBB // 2),
        in_specs=[
            pl.BlockSpec(
                (8, 128, 128), lambda sb, bp: (sb * (_NBB // 2) + bp, 0, 0)
            ),
        ],
        out_specs=pl.BlockSpec(
            (8, 8, 2, 8, 128), lambda sb, bp: (sb, 0, bp, 0, 0)
        ),
        out_shape=jax.ShapeDtypeStruct((200, 8, _NBB, 8, 128), jnp.float32),
        compiler_params=pltpu.CompilerParams(
            dimension_semantics=("arbitrary", "arbitrary")
        ),
    )(rows3)


@jax.jit
def _lookup(xs, table):
    rows = _sc_gather(xs, table)
    return _tc_transpose(rows.reshape(_B // 256, 128, 128))


def kernel(x, table):
    xs = (
        x.astype(jnp.int32)
        .reshape(_NBB // 2, 2, 128, _SB, 8)
        .transpose(3, 0, 4, 2, 1)
    )
    out5 = _lookup(xs.reshape(-1), table)
    return out5.transpose(2, 4, 0, 1, 3).reshape(4096, 200, _D)


# speedup vs baseline: 1.3795x; 1.0404x over previous
import functools

import jax
import jax.numpy as jnp
from jax import lax
from jax.experimental import pallas as pl
from jax.experimental.pallas import tpu as pltpu
from jax.experimental.pallas import tpu_sc as plsc

_D = 64
_SCALE = 8.0
_NC, _NS = 2, 16
_NW = _NC * _NS
_B = 819200
_CHUNK = 512
_NBUF = 2
_SB = 25
_NBB = 32
_K = 5
_SBP = _SB // _K
_BP = _B // _K


def _sc_gather(xs, table, part):
    b_per_w = _BP // _NW
    nchunks = b_per_w // _CHUNK

    mesh = plsc.VectorSubcoreMesh(core_axis_name="c", subcore_axis_name="s")

    @functools.partial(
        pl.kernel,
        out_type=jax.ShapeDtypeStruct((_BP, _D), jnp.float32),
        mesh=mesh,
        scratch_types=[
            pltpu.VMEM((b_per_w,), jnp.int32),
            pltpu.VMEM((_NBUF, _CHUNK, _D), jnp.float32),
            pltpu.SemaphoreType.DMA,
            pltpu.SemaphoreType.DMA,
            pltpu.SemaphoreType.DMA,
            pltpu.SemaphoreType.DMA,
        ],
        compiler_params=pltpu.CompilerParams(use_tc_tiling_on_sc=False),
    )
    def emb(idx_hbm, table_hbm, out_hbm, idx_v, rows_v, g0, g1, s0, s1):
        wid = lax.axis_index("s") * _NC + lax.axis_index("c")
        base = wid * b_per_w
        pltpu.sync_copy(
            idx_hbm.at[pl.ds(part * _BP + base, b_per_w)], idx_v
        )

        gsem = (g0, g1)
        ssem = (s0, s1)

        def start_gather(slot, c):
            pltpu.make_async_copy(
                table_hbm.at[idx_v.at[pl.ds(c * _CHUNK, _CHUNK)]],
                rows_v.at[slot],
                gsem[slot],
            ).start()

        def wait_gather(slot):
            pltpu.make_async_copy(
                table_hbm.at[idx_v.at[pl.ds(0, _CHUNK)]],
                rows_v.at[slot],
                gsem[slot],
            ).wait()

        def start_scatter(slot, c):
            pltpu.make_async_copy(
                rows_v.at[slot],
                out_hbm.at[pl.ds(base + c * _CHUNK, _CHUNK)],
                ssem[slot],
            ).start()

        def wait_scatter(slot):
            pltpu.make_async_copy(
                rows_v.at[slot],
                out_hbm.at[pl.ds(base, _CHUNK)],
                ssem[slot],
            ).wait()

        start_gather(0, 0)
        start_gather(1, 1)

        @pl.loop(0, (nchunks - 2) // 2)
        def _(i):
            c0 = i * 2
            for b in range(_NBUF):
                wait_gather(b)
                start_scatter(b, c0 + b)
            for b in range(_NBUF):
                wait_scatter(b)
                start_gather(b, c0 + 2 + b)

        for b in range(_NBUF):
            wait_gather(b)
            start_scatter(b, nchunks - 2 + b)
        for b in range(_NBUF):
            wait_scatter(b)

    return emb(xs, table)


def _tc_body_first(rows_ref, out_ref):
    for k in range(8):
        pt = jnp.transpose(rows_ref[k], (1, 0)) * _SCALE
        for j in range(2):
            out_ref[k, :, j, :, :] = pt[j * _D : (j + 1) * _D, :].reshape(
                8, 8, 128
            )


def _tc_body(prev_ref, rows_ref, out_ref):
    del prev_ref
    for k in range(8):
        pt = jnp.transpose(rows_ref[k], (1, 0)) * _SCALE
        for j in range(2):
            out_ref[k, :, j, :, :] = pt[j * _D : (j + 1) * _D, :].reshape(
                8, 8, 128
            )


def _tc_transpose(rows3, part, prev):
    out_shape = jax.ShapeDtypeStruct((200, 8, _NBB, 8, 128), jnp.float32)
    grid = (_SBP, _NBB // 2)

    def in_map(sb, bp):
        return (sb * (_NBB // 2) + bp, 0, 0)

    def out_map2(sb, bp, _part=part):
        return (_part * _SBP + sb, 0, bp, 0, 0)

    rows_spec = pl.BlockSpec((8, 128, 128), in_map)
    out_spec = pl.BlockSpec((8, 8, 2, 8, 128), out_map2)
    if prev is None:
        return pl.pallas_call(
            _tc_body_first,
            grid=grid,
            in_specs=[rows_spec],
            out_specs=out_spec,
            out_shape=out_shape,
            compiler_params=pltpu.CompilerParams(
                dimension_semantics=("arbitrary", "arbitrary")
            ),
        )(rows3)
    return pl.pallas_call(
        _tc_body,
        grid=grid,
        in_specs=[pl.BlockSpec(memory_space=pl.ANY), rows_spec],
        out_specs=out_spec,
        out_shape=out_shape,
        input_output_aliases={0: 0},
        compiler_params=pltpu.CompilerParams(
            dimension_semantics=("arbitrary", "arbitrary")
        ),
    )(prev, rows3)


@jax.jit
def _lookup(xs, table):
    rows = [_sc_gather(xs, table, p) for p in range(_K)]
    out = None
    for p in range(_K):
        out = _tc_transpose(rows[p].reshape(_BP // 256, 128, 128), p, out)
    return out


def kernel(x, table):
    xs = (
        x.astype(jnp.int32)
        .reshape(_NBB // 2, 2, 128, _SB, 8)
        .transpose(3, 0, 4, 2, 1)
    )
    out5 = _lookup(xs.reshape(-1), table)
    return out5.transpose(2, 4, 0, 1, 3).reshape(4096, 200, _D)
